# baseline (speedup 1.0000x reference)
import functools

import jax
import jax.numpy as jnp
from jax import lax
from jax.experimental import pallas as pl
from jax.experimental.pallas import tpu as pltpu
from jax.experimental.pallas import tpu_sc as plsc

VOCAB = 16
D = 1024
TOKENS = 4 * 8192

_info = plsc.get_sparse_core_info()
NC, NS = _info.num_cores, _info.num_subcores
NW = NC * NS
TPW = TOKENS // NW
DEPTH = 16


def _build_trc_body(w_ref, cm_ref, trc_ref):
    w = w_ref[...]
    cm = cm_ref[...]
    onehot = (cm == lax.broadcasted_iota(jnp.int32, (VOCAB, VOCAB), 1))
    sel = jax.lax.dot(onehot.astype(jnp.float32), w,
                      precision=jax.lax.Precision.HIGHEST)
    revp = (lax.broadcasted_iota(jnp.int32, (D, D), 0)
            + lax.broadcasted_iota(jnp.int32, (D, D), 1)) == (D - 1)
    trc_ref[...] = jax.lax.dot(sel, revp.astype(jnp.float32),
                               precision=jax.lax.Precision.HIGHEST)


def _build_trc(weight, comp_map):
    return pl.pallas_call(
        _build_trc_body,
        out_shape=jax.ShapeDtypeStruct((VOCAB, D), jnp.float32),
    )(weight, comp_map.reshape(VOCAB, 1))


def _sc_write_body(ids_hbm, w_hbm, trc_hbm, out_hbm, ids_sm, ids_v, tab_v,
                   semw):
    sid = lax.axis_index("s")
    wid = sid * NC + lax.axis_index("c")
    base = wid * TPW

    pltpu.sync_copy(w_hbm, tab_v.at[pl.ds(0, VOCAB), pl.ds(0, D)])
    pltpu.sync_copy(trc_hbm, tab_v.at[pl.ds(0, VOCAB), pl.ds(D, D)])
    pltpu.sync_copy(ids_hbm.at[pl.ds(base, TPW)], ids_v.at[sid])
    pltpu.sync_copy(ids_v.at[sid], ids_sm)

    def issue(t, carry):
        pltpu.async_copy(tab_v.at[ids_sm[t]], out_hbm.at[base + t], semw)
        return carry

    def wait_one():
        pltpu.make_async_copy(tab_v.at[0], out_hbm.at[base], semw).wait()

    def step(t, carry):
        issue(t, carry)
        wait_one()
        return carry

    def drain(j, carry):
        wait_one()
        return carry

    lax.fori_loop(0, DEPTH, issue, 0)
    lax.fori_loop(DEPTH, TPW, step, 0)
    lax.fori_loop(0, DEPTH, drain, 0)


def _sc_write(ids, weight, trc):
    mesh = plsc.VectorSubcoreMesh(core_axis_name="c", subcore_axis_name="s")
    f = functools.partial(
        pl.kernel,
        mesh=mesh,
        out_type=jax.ShapeDtypeStruct((TOKENS, 2 * D), jnp.float32),
        scratch_types=[
            pltpu.SMEM((TPW,), jnp.int32),
            pltpu.VMEM_SHARED((NS, TPW), jnp.int32),
            pltpu.VMEM((VOCAB, 2 * D), jnp.float32),
            pltpu.SemaphoreType.DMA,
        ],
    )(_sc_write_body)
    return f(ids, weight, trc)


def kernel(input_ids, weight, comp_map):
    ids = input_ids.reshape(-1)
    trc = _build_trc(weight, comp_map)
    out = _sc_write(ids, weight, trc)
    return out.reshape(input_ids.shape[0], input_ids.shape[1], 2 * D)
